# Initial kernel scaffold; baseline (speedup 1.0000x reference)
import jax
import jax.numpy as jnp
from jax.experimental import pallas as pl


def kernel(inputs, frequencies, edges):
    raise NotImplementedError("write your pallas kernel here")


# speedup vs baseline: 754.6699x; 754.6699x over previous
import jax
import jax.numpy as jnp
from jax import lax
from jax.experimental import pallas as pl
from jax.experimental.pallas import tpu as pltpu
from jax.experimental.pallas import tpu_sc as plsc

D = 16
NBINS = 5
NC = 2
NS = 16
NW = NC * NS
BLK = 2048


def _body(in_hbm, freq_hbm, edges_hbm, out_hbm, inb, acc, ef, pf, pn):
    n_flat = in_hbm.shape[0]
    rows_per_w = n_flat // (D * NW)
    nblk = rows_per_w // BLK

    wid = lax.axis_index("s") * NC + lax.axis_index("c")

    pltpu.sync_copy(edges_hbm, ef)
    pltpu.sync_copy(freq_hbm, pf)

    rows = [pf[pl.ds(D * k, D)] for k in range(NBINS)]
    total = rows[0] + rows[1] + rows[2] + rows[3] + rows[4]
    inv = 1.0 / total
    for k in range(NBINS):
        pn[pl.ds(D * k, D)] = rows[k] * inv

    iota = lax.iota(jnp.int32, D)
    row0 = wid * rows_per_w

    def block_body(b, carry):
        base = row0 + b * BLK
        pltpu.sync_copy(in_hbm.at[pl.ds(base * D, BLK * D)], inb)
        for d in range(D):
            e = [plsc.load_gather(ef, [jnp.full((D,), D * k + d, jnp.int32)])
                 for k in range(1, NBINS)]
            p = [plsc.load_gather(pn, [jnp.full((D,), D * k + d, jnp.int32)])
                 for k in range(NBINS)]

            def fbody(j, idxv, d=d, e=e, p=p):
                x = plsc.load_gather(inb, [idxv])
                prob = p[0]
                for k in range(1, NBINS):
                    prob = jnp.where(x >= e[k - 1], p[k], prob)
                sl = pl.ds(j * D, D)
                if d == 0:
                    acc[sl] = prob
                else:
                    acc[sl] = acc[sl] * prob
                return idxv + D * D

            lax.fori_loop(0, BLK // D, fbody, iota * D + d)
        pltpu.sync_copy(acc, out_hbm.at[pl.ds(base, BLK)])
        return carry

    lax.fori_loop(0, nblk, block_body, 0)


def kernel(inputs, frequencies, edges):
    n = inputs.shape[0]
    mesh = plsc.VectorSubcoreMesh(
        core_axis_name="c", subcore_axis_name="s",
        num_cores=NC, num_subcores=NS)
    run = pl.kernel(
        _body,
        out_type=jax.ShapeDtypeStruct((n,), jnp.float32),
        mesh=mesh,
        compiler_params=pltpu.CompilerParams(needs_layout_passes=False),
        scratch_types=[
            pltpu.VMEM((BLK * D,), jnp.float32),
            pltpu.VMEM((BLK,), jnp.float32),
            pltpu.VMEM((D * (NBINS + 1),), jnp.float32),
            pltpu.VMEM((D * NBINS,), jnp.float32),
            pltpu.VMEM((D * NBINS,), jnp.float32),
        ],
    )
    return run(inputs.reshape(-1), frequencies.reshape(-1), edges.reshape(-1))
